# baseline (speedup 1.0000x reference)
import functools

import jax
import jax.numpy as jnp
from jax import lax
from jax.experimental import pallas as pl
from jax.experimental.pallas import tpu as pltpu
from jax.experimental.pallas import tpu_sc as plsc

D = 1024
B = 4
S = 2048
EPS = 1e-5
NW = 32
RPW = S // NW
CHG = 32
RPB = 256

_mesh = plsc.VectorSubcoreMesh(core_axis_name="c", subcore_axis_name="s")


@functools.partial(
    pl.kernel,
    mesh=_mesh,
    out_type=jax.ShapeDtypeStruct((S, D), jnp.float32),
    scratch_types=[
        pltpu.VMEM((RPW,), jnp.int32),
        pltpu.VMEM((2 * CHG, D), jnp.float32),
        pltpu.SemaphoreType.DMA((2,)),
        pltpu.SemaphoreType.DMA((2,)),
    ],
)
def _sc_gather(ids_hbm, tok_hbm, out_hbm, idx_v, buf, gsem, ssem):
    wid = lax.axis_index("s") * 2 + lax.axis_index("c")
    r0 = wid * RPW

    pltpu.sync_copy(ids_hbm.at[pl.ds(r0, RPW)], idx_v)

    gathers = []
    for k in range(2):
        gathers.append(pltpu.async_copy(
            tok_hbm.at[idx_v.at[pl.ds(k * CHG, CHG)]],
            buf.at[pl.ds(k * CHG, CHG)], gsem.at[k]))

    stores = []
    for k in range(2):
        gathers[k].wait()
        stores.append(pltpu.async_copy(
            buf.at[pl.ds(k * CHG, CHG)],
            out_hbm.at[pl.ds(r0 + k * CHG, CHG)], ssem.at[k]))
    for k in range(2):
        stores[k].wait()


def _ln_body(t_ref, p_ref, g_ref, b_ref, o_ref):
    x = t_ref[...] + p_ref[...]
    mu = jnp.mean(x, axis=1, keepdims=True)
    xc = x - mu
    var = jnp.mean(xc * xc, axis=1, keepdims=True)
    o_ref[...] = xc * lax.rsqrt(var + EPS) * g_ref[...] + b_ref[...]


_ln_tc = pl.pallas_call(
    _ln_body,
    grid=(S // RPB,),
    in_specs=[
        pl.BlockSpec((RPB, D), lambda i: (i, 0)),
        pl.BlockSpec((RPB, D), lambda i: (i, 0)),
        pl.BlockSpec((1, D), lambda i: (0, 0)),
        pl.BlockSpec((1, D), lambda i: (0, 0)),
    ],
    out_specs=pl.BlockSpec((RPB, D), lambda i: (i, 0)),
    out_shape=jax.ShapeDtypeStruct((S, D), jnp.float32),
)


def kernel(input_ids, tok_table, pos_table, gamma, beta):
    ids = jnp.asarray(input_ids, jnp.int32)
    g2 = gamma.reshape(1, D)
    b2 = beta.reshape(1, D)
    gathered = [_sc_gather(ids[b], tok_table) for b in range(B)]
    normed = [_ln_tc(t, pos_table, g2, b2) for t in gathered]
    return jnp.stack(normed).reshape(B, S, D)
